# baseline (speedup 1.0000x reference)
import functools

import jax
import jax.numpy as jnp
from jax.experimental import pallas as pl
from jax.experimental.pallas import tpu as pltpu

B, D, K_NEG, N_HARD = 1024, 256, 65536, 1024
N1 = N2 = N3 = 128
N_SYN = N1 + N2 + N3
TEMP = 0.2
HARD_ALPHA, HARD_BETA, HARD_GAMMA = 0.5, 1.5, 1.0
OUT_W = K_NEG + N_SYN

RB = 128
CB = 8192
NI = B // RB
NJ = K_NEG // CB
INV_TEMP = 1.0 / TEMP


def _logits_body(q_ref, w_ref, o_ref, colnorm_ref):
    i = pl.program_id(1)

    @pl.when(i == 0)
    def _():
        w = w_ref[...]
        colnorm_ref[...] = jnp.maximum(
            jnp.sqrt(jnp.sum(w * w, axis=0, keepdims=True)), 1e-12)

    q = q_ref[...]
    qn = q / jnp.maximum(
        jnp.sqrt(jnp.sum(q * q, axis=1, keepdims=True)), 1e-12)
    acc = jax.lax.dot_general(
        qn, w_ref[...], (((1,), (0,)), ((), ())),
        preferred_element_type=jnp.float32,
        precision=jax.lax.Precision.HIGHEST)
    o_ref[...] = acc * (INV_TEMP / colnorm_ref[...])


def _logits_call(q, queue):
    return pl.pallas_call(
        _logits_body,
        grid=(NJ, NI),
        in_specs=[
            pl.BlockSpec((RB, D), lambda j, i: (i, 0)),
            pl.BlockSpec((D, CB), lambda j, i: (0, j)),
        ],
        out_specs=pl.BlockSpec((RB, CB), lambda j, i: (i, j)),
        out_shape=jax.ShapeDtypeStruct((B, OUT_W), jnp.float32),
        scratch_shapes=[pltpu.VMEM((1, CB), jnp.float32)],
    )(q, queue)


def _lhard_body(out_in_ref, c1_ref, a_ref, c2_ref, b_ref, c3a_ref, c3b_ref,
                g_ref, d_ref, o_ref):
    del out_in_ref
    k = pl.program_id(1)
    c1 = c1_ref[...]
    a = a_ref[...]
    l1 = (a + (1.0 - a) * c1) * jax.lax.rsqrt(
        a * a + (1.0 - a) * (1.0 - a) + 2.0 * a * (1.0 - a) * c1)
    c2 = c2_ref[...]
    b = b_ref[...]
    u = 1.0 - b
    l2 = (u + b * c2) * jax.lax.rsqrt(u * u + b * b + 2.0 * u * b * c2)
    c3a = c3a_ref[...]
    c3b = c3b_ref[...]
    g = g_ref[...]
    d = d_ref[...]
    l3 = (g * c3a + (1.0 - g) * c3b) * jax.lax.rsqrt(
        g * g + (1.0 - g) * (1.0 - g) + 2.0 * g * (1.0 - g) * d)
    o_ref[...] = jnp.where(k == 0, l1, jnp.where(k == 1, l2, l3)) * INV_TEMP


def _lhard_call(out_main, c1, a, c2, b, c3a, c3b, g, d):
    small = lambda: pl.BlockSpec((RB, 128), lambda i, k: (i, 0))
    return pl.pallas_call(
        _lhard_body,
        grid=(NI, 3),
        in_specs=[pl.BlockSpec(memory_space=pl.ANY)] + [small() for _ in range(8)],
        out_specs=pl.BlockSpec((RB, 128), lambda i, k: (i, K_NEG // 128 + k)),
        out_shape=jax.ShapeDtypeStruct((B, OUT_W), jnp.float32),
        input_output_aliases={0: 0},
    )(out_main, c1, a, c2, b, c3a, c3b, g, d)


def _fixed_constants():
    rk = jax.random.key(42)
    k_i1, k_a1, k_i2, k_b2, k_i3a, k_i3b, k_g3 = jax.random.split(rk, 7)
    idxs1 = jax.random.randint(k_i1, (B, N1), 0, N_HARD)
    alpha = jax.random.uniform(k_a1, (B, N1, 1), dtype=jnp.float32)[..., 0] * HARD_ALPHA
    idxs2 = jax.random.randint(k_i2, (B, N2), 0, N_HARD)
    beta = 1.0 + jax.random.uniform(k_b2, (B, N2, 1), dtype=jnp.float32)[..., 0] * (
        HARD_BETA - 1.0)
    idxs3a = jax.random.randint(k_i3a, (B, N3), 0, N_HARD)
    idxs3b = jax.random.randint(k_i3b, (B, N3), 0, N_HARD)
    gamma = jax.random.uniform(k_g3, (B, N3, 1), dtype=jnp.float32)[..., 0] * HARD_GAMMA
    return idxs1, alpha, idxs2, beta, idxs3a, idxs3b, gamma


def kernel(q, queue):
    out_main = _logits_call(q, queue)
    lv = out_main[:, :K_NEG]
    vals, top_idx = jax.lax.top_k(lv, N_HARD)
    c_sorted = vals * TEMP

    idxs1, alpha, idxs2, beta, idxs3a, idxs3b, gamma = _fixed_constants()

    c1 = jnp.take_along_axis(c_sorted, idxs1, axis=1)
    c2 = jnp.take_along_axis(c_sorted, idxs2, axis=1)
    c3a = jnp.take_along_axis(c_sorted, idxs3a, axis=1)
    c3b = jnp.take_along_axis(c_sorted, idxs3b, axis=1)

    sel3a = jnp.take_along_axis(top_idx, idxs3a, axis=1)
    sel3b = jnp.take_along_axis(top_idx, idxs3b, axis=1)
    qt = queue.T
    colnorm = jnp.maximum(jnp.linalg.norm(queue, axis=0), 1e-12)
    na = qt[sel3a] / colnorm[sel3a][..., None]
    nb = qt[sel3b] / colnorm[sel3b][..., None]
    d = jnp.sum(na * nb, axis=-1)

    return _lhard_call(out_main, c1, alpha, c2, beta, c3a, c3b, gamma, d)


# speedup vs baseline: 3.1159x; 3.1159x over previous
import functools

import jax
import jax.numpy as jnp
from jax import lax
from jax.experimental import pallas as pl
from jax.experimental.pallas import tpu as pltpu
from jax.experimental.pallas import tpu_sc as plsc

B, D, K_NEG, N_HARD = 1024, 256, 65536, 1024
N1 = N2 = N3 = 128
N_SYN = N1 + N2 + N3
TEMP = 0.2
HARD_ALPHA, HARD_BETA, HARD_GAMMA = 0.5, 1.5, 1.0
OUT_W = K_NEG + N_SYN

RB = 128
CB = 4096
NI = B // RB
NJ = K_NEG // CB
INV_TEMP = 1.0 / TEMP

CAP = 2048
CNT_TARGET = 1536


def _logits_body(q_ref, w_ref, o_ref, qtn_ref, colnorm_ref, cn_ref):
    i = pl.program_id(1)

    @pl.when(i == 0)
    def _():
        w = w_ref[...]
        cn = jnp.maximum(jnp.sqrt(jnp.sum(w * w, axis=0, keepdims=True)), 1e-12)
        cn_ref[...] = cn
        colnorm_ref[...] = cn
        qtn_ref[...] = w.T / cn.T

    q = q_ref[...]
    qn = q / jnp.maximum(
        jnp.sqrt(jnp.sum(q * q, axis=1, keepdims=True)), 1e-12)
    acc = jax.lax.dot_general(
        qn, w_ref[...], (((1,), (0,)), ((), ())),
        preferred_element_type=jnp.float32,
        precision=jax.lax.Precision.HIGHEST)
    o_ref[...] = acc * (INV_TEMP / cn_ref[...])


def _logits_call(q, queue):
    return pl.pallas_call(
        _logits_body,
        grid=(NJ, NI),
        in_specs=[
            pl.BlockSpec((RB, D), lambda j, i: (i, 0)),
            pl.BlockSpec((D, CB), lambda j, i: (0, j)),
        ],
        out_specs=[
            pl.BlockSpec((RB, CB), lambda j, i: (i, j)),
            pl.BlockSpec((CB, D), lambda j, i: (j, 0)),
            pl.BlockSpec((1, CB), lambda j, i: (0, j)),
        ],
        out_shape=[
            jax.ShapeDtypeStruct((B, OUT_W), jnp.float32),
            jax.ShapeDtypeStruct((K_NEG, D), jnp.float32),
            jax.ShapeDtypeStruct((1, K_NEG), jnp.float32),
        ],
        scratch_shapes=[pltpu.VMEM((1, CB), jnp.float32)],
    )(q, queue)


RBB = 8


def _bisect_body(o_ref, thr_ref, cnt_ref):
    x = o_ref[...]
    u = pltpu.bitcast(x, jnp.uint32)
    key = u ^ jnp.where(u >> 31 == 0, jnp.uint32(0x80000000),
                        jnp.uint32(0xFFFFFFFF))

    def count_ge(t):
        return jnp.sum((key >= t).astype(jnp.int32), axis=1, keepdims=True)

    def cond(carry):
        _, width, cnt = carry
        return jnp.logical_and(width > 0, jnp.any(cnt > CNT_TARGET))

    def body(carry):
        lo, width, cnt = carry
        mid = lo + width
        c = count_ge(mid)
        ok = jnp.logical_and(mid > lo, c >= N_HARD)
        lo = jnp.where(ok, mid, lo)
        cnt = jnp.where(ok, c, cnt)
        return lo, width // 2, cnt

    lo0 = jnp.zeros((RBB, 1), jnp.uint32)
    cnt0 = jnp.full((RBB, 1), K_NEG, jnp.int32)
    lo, _, cnt = jax.lax.while_loop(
        cond, body, (lo0, jnp.uint32(1 << 31), cnt0))
    thr_u = jnp.where(lo >> 31 == 1, lo ^ jnp.uint32(0x80000000), ~lo)
    thr = pltpu.bitcast(thr_u, jnp.float32)
    thr_ref[...] = jnp.broadcast_to(thr, (RBB, 128))
    cnt_ref[...] = jnp.broadcast_to(cnt, (RBB, 128))


def _bisect_call(out_main):
    return pl.pallas_call(
        _bisect_body,
        grid=(B // RBB,),
        in_specs=[pl.BlockSpec((RBB, K_NEG), lambda i: (i, 0))],
        out_specs=[
            pl.BlockSpec((RBB, 128), lambda i: (i, 0)),
            pl.BlockSpec((RBB, 128), lambda i: (i, 0)),
        ],
        out_shape=[
            jax.ShapeDtypeStruct((B, 128), jnp.float32),
            jax.ShapeDtypeStruct((B, 128), jnp.int32),
        ],
    )(out_main)


def _compact_xla(out_main, thr, cnt):
    lv = out_main[:, :K_NEG]
    mask = lv >= thr[:, :1]
    order = jnp.argsort(~mask, axis=1, stable=True)
    cand_idx = order[:, :CAP].astype(jnp.int32)
    cand_val = jnp.take_along_axis(lv, cand_idx, axis=1)
    return cand_val, cand_idx


WINW = 8192
NWIN = K_NEG // WINW


def _compact_call(out_main, thr1):
    info = plsc.get_sparse_core_info()
    NC, NS = info.num_cores, info.num_subcores
    NW = NC * NS
    RPW = B // NW
    mesh = plsc.VectorSubcoreMesh(core_axis_name="c", subcore_axis_name="s")

    @functools.partial(
        pl.kernel, mesh=mesh,
        out_type=[
            jax.ShapeDtypeStruct((B, CAP), jnp.float32),
            jax.ShapeDtypeStruct((B, CAP), jnp.int32),
        ],
        scratch_types=[
            pltpu.VMEM((RPW * 16,), jnp.float32),
            pltpu.VMEM((WINW,), jnp.float32),
            pltpu.VMEM((WINW,), jnp.float32),
            pltpu.VMEM((CAP,), jnp.float32),
            pltpu.VMEM((CAP,), jnp.int32),
            pltpu.SemaphoreType.DMA,
            pltpu.SemaphoreType.DMA,
        ],
    )
    def compact(out_hbm, thr_hbm, cval_hbm, cidx_hbm,
                thr_ts, win0, win1, cval, cidx, sem0, sem1):
        wid = lax.axis_index("s") * NC + lax.axis_index("c")
        base = wid * RPW
        pltpu.sync_copy(thr_hbm.at[pl.ds(base * 16, RPW * 16)], thr_ts)
        iota16 = lax.iota(jnp.int32, 16)
        wins = (win0, win1)
        sems = (sem0, sem1)

        def do_row(r, _):
            row = base + r
            thr_v = thr_ts[pl.ds(pl.multiple_of(r * 16, 16), 16)]
            cps = {0: pltpu.async_copy(
                out_hbm.at[row, pl.ds(0, WINW)], win0, sem0)}
            off = jnp.int32(0)
            for w in range(NWIN):
                if w + 1 < NWIN:
                    cps[w + 1] = pltpu.async_copy(
                        out_hbm.at[row, pl.ds((w + 1) * WINW, WINW)],
                        wins[(w + 1) % 2], sems[(w + 1) % 2])
                cps[w].wait()
                buf = wins[w % 2]

                def vstep(k, off, w=w, buf=buf):
                    v = buf[pl.ds(pl.multiple_of(k * 16, 16), 16)]
                    mb = v >= thr_v
                    m = mb.astype(jnp.int32)
                    tot = plsc.all_reduce_population_count(mb)[0]

                    @pl.when(tot > 0)
                    def _():
                        key = iota16 + (1 - m) * 16
                        colv = iota16 + (w * WINW + k * 16)
                        _, sv = plsc.sort_key_val(key, v)
                        _, si = plsc.sort_key_val(key, colv)
                        pos = off + iota16
                        mask = jnp.logical_and(iota16 < tot, pos < CAP)
                        plsc.store_scatter(cval, [pos], sv, mask=mask)
                        plsc.store_scatter(cidx, [pos], si, mask=mask)

                    return off + tot

                off = lax.fori_loop(0, WINW // 16, vstep, off)
            pltpu.sync_copy(cval, cval_hbm.at[row])
            pltpu.sync_copy(cidx, cidx_hbm.at[row])
            return 0

        lax.fori_loop(0, RPW, do_row, 0)

    return compact(out_main, thr1)


RBD = 8
_STAGES = []
for _k in [2 << s for s in range(11)]:
    _j = _k // 2
    while _j >= 1:
        _STAGES.append((_k, _j))
        _j //= 2


def _sort_body(cv_ref, ci_ref, cnt_ref, sv_ref, si_ref):
    iota = jax.lax.broadcasted_iota(jnp.int32, (RBD, CAP), 1)
    cnt = cnt_ref[...][:, :1]
    valid = iota < cnt
    v = jnp.where(valid, cv_ref[...], -jnp.inf)
    ix = jnp.where(valid, ci_ref[...], jnp.int32(0x3FFFFFFF))

    def stage(k, j, carry):
        v, ix = carry
        upper = (iota & j) != 0
        pv = jnp.where(upper, pltpu.roll(v, j, 1), pltpu.roll(v, -j, 1))
        pi = jnp.where(upper, pltpu.roll(ix, j, 1), pltpu.roll(ix, -j, 1))
        desc = (iota & k) == 0
        take_max = desc != upper
        p_gt = jnp.logical_or(pv > v, jnp.logical_and(pv == v, pi < ix))
        sel_p = take_max == p_gt
        v = jnp.where(sel_p, pv, v)
        ix = jnp.where(sel_p, pi, ix)
        return v, ix

    def phase(p, carry):
        k = jnp.int32(1) << p

        def inner(m, carry):
            j = jnp.int32(1) << (p - 1 - m)
            return stage(k, j, carry)

        return jax.lax.fori_loop(0, p, inner, carry)

    v, ix = jax.lax.fori_loop(1, 12, phase, (v, ix))
    sv_ref[...] = v[:, :N_HARD]
    si_ref[...] = ix[:, :N_HARD]


def _sort_call(cand_val, cand_idx, cnt):
    return pl.pallas_call(
        _sort_body,
        grid=(B // RBD,),
        in_specs=[
            pl.BlockSpec((RBD, CAP), lambda i: (i, 0)),
            pl.BlockSpec((RBD, CAP), lambda i: (i, 0)),
            pl.BlockSpec((RBD, 128), lambda i: (i, 0)),
        ],
        out_specs=[
            pl.BlockSpec((RBD, N_HARD), lambda i: (i, 0)),
            pl.BlockSpec((RBD, N_HARD), lambda i: (i, 0)),
        ],
        out_shape=[
            jax.ShapeDtypeStruct((B, N_HARD), jnp.float32),
            jax.ShapeDtypeStruct((B, N_HARD), jnp.int32),
        ],
    )(cand_val, cand_idx, cnt)


def _gather_xla(svals, sidx, qtn, ranks):
    c = jnp.take_along_axis(svals, ranks, axis=1)
    c1, c2, c3a, c3b = jnp.split(c, 4, axis=1)
    sel3a = jnp.take_along_axis(sidx, ranks[:, 256:384], axis=1)
    sel3b = jnp.take_along_axis(sidx, ranks[:, 384:], axis=1)
    d = jnp.sum(qtn[sel3a] * qtn[sel3b], axis=-1)
    return c1, c2, c3a, c3b, d


def _gather_call(svals, sidx, qtn, ranks):
    info = plsc.get_sparse_core_info()
    NC, NS = info.num_cores, info.num_subcores
    NW = NC * NS
    RPW = B // NW
    mesh = plsc.VectorSubcoreMesh(core_axis_name="c", subcore_axis_name="s")

    svals_flat = svals.reshape(B * N_HARD)
    sidx_flat = sidx.reshape(B * N_HARD)
    ranks_abs = ranks + (jnp.arange(B, dtype=jnp.int32) * N_HARD)[:, None]
    ra1 = ranks_abs[:, :128]
    ra2 = ranks_abs[:, 128:256]
    ra3a = ranks_abs[:, 256:384]
    ra3b = ranks_abs[:, 384:]

    @functools.partial(
        pl.kernel, mesh=mesh,
        out_type=[
            jax.ShapeDtypeStruct((B, 512), jnp.float32),
            jax.ShapeDtypeStruct((B, 2048), jnp.float32),
        ],
        scratch_types=[
            pltpu.VMEM((128,), jnp.int32),
            pltpu.VMEM((128,), jnp.int32),
            pltpu.VMEM((128,), jnp.int32),
            pltpu.VMEM((128,), jnp.int32),
            pltpu.VMEM((512,), jnp.float32),
            pltpu.VMEM((128,), jnp.int32),
            pltpu.VMEM((128,), jnp.int32),
            pltpu.VMEM((128, D), jnp.float32),
            pltpu.VMEM((128, D), jnp.float32),
            pltpu.VMEM((2048,), jnp.float32),
            pltpu.SemaphoreType.DMA,
        ],
    )
    def gatherk(svals_hbm, sidx_hbm, ra1_hbm, ra2_hbm, ra3a_hbm, ra3b_hbm,
                qtn_hbm, cg_hbm, d_hbm,
                r1_ts, r2_ts, r3a_ts, r3b_ts, csel_ts, sela_ts, selb_ts,
                colsa, colsb, outd, sem):
        wid = lax.axis_index("s") * NC + lax.axis_index("c")
        base = wid * RPW
        iota16 = lax.iota(jnp.int32, 16)

        def do_row(r, carry):
            row = base + r
            pltpu.sync_copy(ra1_hbm.at[row], r1_ts)
            pltpu.sync_copy(ra2_hbm.at[row], r2_ts)
            pltpu.sync_copy(ra3a_hbm.at[row], r3a_ts)
            pltpu.sync_copy(ra3b_hbm.at[row], r3b_ts)
            pltpu.sync_copy(svals_hbm.at[r1_ts], csel_ts.at[pl.ds(0, 128)])
            pltpu.sync_copy(svals_hbm.at[r2_ts], csel_ts.at[pl.ds(128, 128)])
            pltpu.sync_copy(svals_hbm.at[r3a_ts], csel_ts.at[pl.ds(256, 128)])
            pltpu.sync_copy(svals_hbm.at[r3b_ts], csel_ts.at[pl.ds(384, 128)])
            pltpu.sync_copy(sidx_hbm.at[r3a_ts], sela_ts)
            pltpu.sync_copy(sidx_hbm.at[r3b_ts], selb_ts)
            cpa = pltpu.async_copy(qtn_hbm.at[sela_ts], colsa, sem)
            cpa.wait()
            cpb = pltpu.async_copy(qtn_hbm.at[selb_ts], colsb, sem)
            cpb.wait()

            def pair(p, c):
                acc = jnp.zeros((16,), jnp.float32)
                for kk in range(D // 16):
                    va = colsa[p, pl.ds(16 * kk, 16)]
                    vb = colsb[p, pl.ds(16 * kk, 16)]
                    acc = acc + va * vb
                for j in (1, 2, 4, 8):
                    acc = acc + acc[(iota16 ^ j,)]
                outd[pl.ds(pl.multiple_of(p * 16, 16), 16)] = acc
                return c

            lax.fori_loop(0, 128, pair, 0)
            pltpu.sync_copy(csel_ts, cg_hbm.at[row])
            pltpu.sync_copy(outd, d_hbm.at[row])
            return carry

        lax.fori_loop(0, RPW, do_row, 0)

    cg, dwide = gatherk(svals_flat, sidx_flat, ra1, ra2, ra3a, ra3b, qtn)
    return cg, dwide[:, ::16]


def _lhard_body(out_in_ref, c1_ref, a_ref, c2_ref, b_ref, c3a_ref, c3b_ref,
                g_ref, d_ref, o_ref):
    del out_in_ref
    k = pl.program_id(1)
    c1 = c1_ref[...] * TEMP
    a = a_ref[...]
    l1 = (a + (1.0 - a) * c1) * jax.lax.rsqrt(
        a * a + (1.0 - a) * (1.0 - a) + 2.0 * a * (1.0 - a) * c1)
    c2 = c2_ref[...] * TEMP
    b = b_ref[...]
    u = 1.0 - b
    l2 = (u + b * c2) * jax.lax.rsqrt(u * u + b * b + 2.0 * u * b * c2)
    c3a = c3a_ref[...] * TEMP
    c3b = c3b_ref[...] * TEMP
    g = g_ref[...]
    d = d_ref[...]
    l3 = (g * c3a + (1.0 - g) * c3b) * jax.lax.rsqrt(
        g * g + (1.0 - g) * (1.0 - g) + 2.0 * g * (1.0 - g) * d)
    o_ref[...] = jnp.where(k == 0, l1, jnp.where(k == 1, l2, l3)) * INV_TEMP


def _lhard_call(out_main, c1, a, c2, b, c3a, c3b, g, d):
    small = lambda: pl.BlockSpec((RB, 128), lambda i, k: (i, 0))
    return pl.pallas_call(
        _lhard_body,
        grid=(NI, 3),
        in_specs=[pl.BlockSpec(memory_space=pl.ANY)] + [small() for _ in range(8)],
        out_specs=pl.BlockSpec((RB, 128), lambda i, k: (i, K_NEG // 128 + k)),
        out_shape=jax.ShapeDtypeStruct((B, OUT_W), jnp.float32),
        input_output_aliases={0: 0},
    )(out_main, c1, a, c2, b, c3a, c3b, g, d)


def _fixed_constants():
    rk = jax.random.key(42)
    k_i1, k_a1, k_i2, k_b2, k_i3a, k_i3b, k_g3 = jax.random.split(rk, 7)
    idxs1 = jax.random.randint(k_i1, (B, N1), 0, N_HARD)
    alpha = jax.random.uniform(k_a1, (B, N1, 1), dtype=jnp.float32)[..., 0] * HARD_ALPHA
    idxs2 = jax.random.randint(k_i2, (B, N2), 0, N_HARD)
    beta = 1.0 + jax.random.uniform(k_b2, (B, N2, 1), dtype=jnp.float32)[..., 0] * (
        HARD_BETA - 1.0)
    idxs3a = jax.random.randint(k_i3a, (B, N3), 0, N_HARD)
    idxs3b = jax.random.randint(k_i3b, (B, N3), 0, N_HARD)
    gamma = jax.random.uniform(k_g3, (B, N3, 1), dtype=jnp.float32)[..., 0] * HARD_GAMMA
    return idxs1, alpha, idxs2, beta, idxs3a, idxs3b, gamma


def _compact_bridge(out_main, thr):
    lv = out_main[:, :K_NEG]
    mask = lv >= thr[:, :1]
    cs = jnp.cumsum(mask.astype(jnp.int32), axis=1)
    targets = jnp.arange(1, CAP + 1, dtype=jnp.int32)
    cand_pos = jax.vmap(
        lambda c: jnp.searchsorted(c, targets, side="left"))(cs)
    cand_idx = jnp.minimum(cand_pos, K_NEG - 1).astype(jnp.int32)
    cand_val = jnp.take_along_axis(lv, cand_idx, axis=1)
    return cand_val, cand_idx


def kernel(q, queue):
    out_main, qtn, _colnorm = _logits_call(q, queue)
    thr, cnt = _bisect_call(out_main)
    cand_val, cand_idx = _compact_bridge(out_main, thr)
    svals, sidx = _sort_call(cand_val, cand_idx, cnt)

    idxs1, alpha, idxs2, beta, idxs3a, idxs3b, gamma = _fixed_constants()
    ranks = jnp.concatenate([idxs1, idxs2, idxs3a, idxs3b], axis=1)
    cg, d = _gather_call(svals, sidx, qtn, ranks)
    c1 = cg[:, :128]
    c2 = cg[:, 128:256]
    c3a = cg[:, 256:384]
    c3b = cg[:, 384:]

    return _lhard_call(out_main, c1, alpha, c2, beta, c3a, c3b, gamma, d)
